# baseline (speedup 1.0000x reference)
import functools

import jax
import jax.numpy as jnp
from jax.experimental import pallas as pl
from jax.experimental.pallas import tpu as pltpu

_CHUNK = 4096
_SELU_SCALE = 1.0507009873554805
_SELU_ALPHA = 1.6732632423543772


def _leaky(v):
    return jnp.where(v >= 0, v, 0.2 * v)


def _selu(v):
    return _SELU_SCALE * jnp.where(v > 0, v, _SELU_ALPHA * (jnp.exp(jnp.minimum(v, 0.0)) - 1.0))


def _dense_body(x_ref, w_ref, as_ref, ad_ref, h_ref, av_ref, dv_ref, b_ref):
    h = jnp.dot(x_ref[...], w_ref[...], preferred_element_type=jnp.float32)
    h_ref[...] = h
    av = jnp.dot(h, as_ref[...], preferred_element_type=jnp.float32)
    dv = jnp.dot(h, ad_ref[...], preferred_element_type=jnp.float32)
    av_ref[...] = av
    dv_ref[...] = dv
    m = jnp.max(av, axis=0, keepdims=True) + jnp.max(dv, axis=0, keepdims=True)
    b_ref[...] = _leaky(m)


def _edge_body(src_ref, dst_ref, av_ref, dv_ref, bnd_ref, h_ref, den_ref, acc_ref):
    @pl.when(pl.program_id(0) == 0)
    def _init():
        den_ref[...] = jnp.zeros_like(den_ref)
        acc_ref[...] = jnp.zeros_like(acc_ref)

    bnd = bnd_ref[0:1, :]

    def body(i, _):
        s = src_ref[0, 0, i]
        d = dst_ref[0, 0, i]
        e = _leaky(av_ref[pl.ds(s, 1), :] + dv_ref[pl.ds(d, 1), :])
        w = jnp.exp(e - bnd)
        den_ref[pl.ds(d, 1), :] += w
        acc_ref[pl.ds(d, 1), :] += w * h_ref[pl.ds(s, 1), :]
        return 0

    jax.lax.fori_loop(0, src_ref.shape[2], body, 0)


def _finalize_body(acc_ref, den_ref, b_ref, out_ref):
    n = out_ref.shape[0]
    out_ref[...] = _selu(acc_ref[0:n, :] / (den_ref[0:n, :] + 1e-16) + b_ref[...])


def _gat_layer(xin, src_c, dst_c, W, a_s, a_d, b):
    n, _ = xin.shape
    f_out = W.shape[1]
    n_pad = n + 8
    h, av, dv, bnd = pl.pallas_call(
        _dense_body,
        out_shape=(
            jax.ShapeDtypeStruct((n, f_out), jnp.float32),
            jax.ShapeDtypeStruct((n, 1), jnp.float32),
            jax.ShapeDtypeStruct((n, 1), jnp.float32),
            jax.ShapeDtypeStruct((1, 1), jnp.float32),
        ),
    )(xin, W, a_s.reshape(f_out, 1), a_d.reshape(f_out, 1))

    zpad = jnp.zeros((8, 1), jnp.float32)
    avp = jnp.concatenate([av, zpad], axis=0)
    dvp = jnp.concatenate([dv, zpad], axis=0)

    nchunks = src_c.shape[0]
    den, acc = pl.pallas_call(
        _edge_body,
        grid=(nchunks,),
        in_specs=[
            pl.BlockSpec((1, 1, _CHUNK), lambda i: (i, 0, 0), memory_space=pltpu.SMEM),
            pl.BlockSpec((1, 1, _CHUNK), lambda i: (i, 0, 0), memory_space=pltpu.SMEM),
            pl.BlockSpec((n_pad, 1), lambda i: (0, 0)),
            pl.BlockSpec((n_pad, 1), lambda i: (0, 0)),
            pl.BlockSpec((1, 1), lambda i: (0, 0)),
            pl.BlockSpec((n, f_out), lambda i: (0, 0)),
        ],
        out_specs=(
            pl.BlockSpec((n_pad, 1), lambda i: (0, 0)),
            pl.BlockSpec((n_pad, f_out), lambda i: (0, 0)),
        ),
        out_shape=(
            jax.ShapeDtypeStruct((n_pad, 1), jnp.float32),
            jax.ShapeDtypeStruct((n_pad, f_out), jnp.float32),
        ),
    )(src_c, dst_c, avp, dvp, bnd, h)

    out = pl.pallas_call(
        _finalize_body,
        out_shape=jax.ShapeDtypeStruct((n, f_out), jnp.float32),
    )(acc, den, b.reshape(1, f_out))
    return out


def _tail_body(h2_ref, batch_ref, x_ref, fidx_ref, w0_ref, b0_ref, wl1_ref,
               bl1_ref, wl2_ref, bl2_ref, out_ref, pooled_ref, news_ref, cat_ref):
    g_cnt = out_ref.shape[0]

    def pool_body(g, _):
        mask = batch_ref[...] == g
        val = jnp.max(jnp.where(mask, h2_ref[...], -jnp.inf), axis=0, keepdims=True)
        pooled_ref[pl.ds(g, 1), :] = val
        return 0

    jax.lax.fori_loop(0, g_cnt, pool_body, 0)

    def news_body(g, _):
        fi = fidx_ref[0, g]
        news_ref[pl.ds(g, 1), :] = x_ref[pl.ds(fi, 1), :]
        return 0

    jax.lax.fori_loop(0, g_cnt, news_body, 0)

    pooled = _selu(pooled_ref[...])
    xg = _selu(jnp.dot(pooled, wl1_ref[...], preferred_element_type=jnp.float32)
               + bl1_ref[...])
    news = jnp.maximum(
        jnp.dot(news_ref[...], w0_ref[...], preferred_element_type=jnp.float32)
        + b0_ref[...], 0.0)
    hdim = xg.shape[1]
    cat_ref[:, 0:hdim] = xg
    cat_ref[:, hdim:2 * hdim] = news
    xg2 = jnp.maximum(
        jnp.dot(cat_ref[...], wl1_ref[...], preferred_element_type=jnp.float32)
        + bl1_ref[...], 0.0)
    logits = jnp.dot(xg2, wl2_ref[...], preferred_element_type=jnp.float32) + bl2_ref[...]
    m = jnp.max(logits, axis=1, keepdims=True)
    z = logits - m
    out_ref[...] = z - jnp.log(jnp.sum(jnp.exp(z), axis=1, keepdims=True))


def kernel(x, edge_index, batch, W1, a_src1, a_dst1, b1, W2, a_src2, a_dst2, b2,
           W0, b0, Wl1, bl1, Wl2, bl2):
    n, f_in = x.shape
    e_cnt = edge_index.shape[1]
    g_cnt = 64
    hdim = W0.shape[1]

    loop = jnp.arange(n, dtype=edge_index.dtype)
    src = jnp.concatenate([edge_index[0], loop])
    dst = jnp.concatenate([edge_index[1], loop])
    e_tot = e_cnt + n
    nchunks = -(-e_tot // _CHUNK)
    pad = nchunks * _CHUNK - e_tot
    src = jnp.concatenate([src, jnp.zeros((pad,), src.dtype)])
    dst = jnp.concatenate([dst, jnp.full((pad,), n, dst.dtype)])
    src_c = src.reshape(nchunks, 1, _CHUNK)
    dst_c = dst.reshape(nchunks, 1, _CHUNK)

    h1 = _gat_layer(x, src_c, dst_c, W1, a_src1, a_dst1, b1)
    h2 = _gat_layer(h1, src_c, dst_c, W2, a_src2, a_dst2, b2)

    first_idx = jnp.searchsorted(batch, jnp.arange(g_cnt, dtype=batch.dtype)
                                 ).astype(jnp.int32).reshape(1, g_cnt)

    out = pl.pallas_call(
        _tail_body,
        in_specs=[
            pl.BlockSpec(memory_space=pltpu.VMEM),
            pl.BlockSpec(memory_space=pltpu.VMEM),
            pl.BlockSpec(memory_space=pltpu.VMEM),
            pl.BlockSpec(memory_space=pltpu.SMEM),
            pl.BlockSpec(memory_space=pltpu.VMEM),
            pl.BlockSpec(memory_space=pltpu.VMEM),
            pl.BlockSpec(memory_space=pltpu.VMEM),
            pl.BlockSpec(memory_space=pltpu.VMEM),
            pl.BlockSpec(memory_space=pltpu.VMEM),
            pl.BlockSpec(memory_space=pltpu.VMEM),
        ],
        out_shape=jax.ShapeDtypeStruct((g_cnt, Wl2.shape[1]), jnp.float32),
        scratch_shapes=[
            pltpu.VMEM((g_cnt, 2 * hdim), jnp.float32),
            pltpu.VMEM((g_cnt, hdim), jnp.float32),
            pltpu.VMEM((g_cnt, 2 * hdim), jnp.float32),
        ],
    )(h2, batch.reshape(n, 1), x, first_idx, W0, b0.reshape(1, hdim),
      Wl1, bl1.reshape(1, hdim), Wl2, bl2.reshape(1, Wl2.shape[1]))
    return out
